# baseline (speedup 1.0000x reference)
import jax
import jax.numpy as jnp
from jax import lax
from jax.experimental import pallas as pl
from jax.experimental.pallas import tpu as pltpu
from jax.experimental.pallas import tpu_sc as plsc

B = 64
S = 4
D = 8
R = S ** D
NC = 2
NS = 16
NW = NC * NS
PER_W = 2 * B // NW

_DNUMS = lax.GatherDimensionNumbers(
    offset_dims=(), collapsed_slice_dims=(0,), start_index_map=(0,))


def _take(v, idx):
    return lax.gather(v, idx[:, None], _DNUMS, (1,),
                      mode=lax.GatherScatterMode.PROMISE_IN_BOUNDS)


CH = 8192
NCHUNK = R // CH


def _sc_body(l_hbm, out_l, in_v, buf0, buf1, sem0, sem1):
    cid = lax.axis_index("c")
    sid = lax.axis_index("s")
    wid = sid * NC + cid

    iota = lax.iota(jnp.int32, 16)
    hi = iota >> 2
    lo = iota & 3
    zeros = jnp.zeros((16,), jnp.int32)

    bufs = (buf0, buf1)
    sems = (sem0, sem1)
    pending = [None, None]
    gch = 0

    for x_hbm, o_hbm in ((l_hbm, out_l),):
      for k in range(B // NW):
        b = wid + NW * k

        pltpu.sync_copy(x_hbm.at[b], in_v)
        vlow = in_v[pl.ds(0, 16)]
        vhigh = in_v[pl.ds(16, 16)]

        def fac(sv, d):
            idx = ((sv & 1) << 3) + d
            return jnp.where(sv < 2, _take(vlow, idx), _take(vhigh, idx))

        t01 = fac(hi, 0) * fac(lo, 1)
        t23 = fac(hi, 2) * fac(lo, 3)
        t45 = fac(hi, 4) * fac(lo, 5)
        t67 = fac(hi, 6) * fac(lo, 7)

        c_regs = [_take(t45, jnp.full((16,), j, jnp.int32)) * t67
                  for j in range(16)]

        for ch in range(NCHUNK):
            p = gch & 1
            gch += 1
            buf = bufs[p]
            if pending[p] is not None:
                pending[p].wait()

            def chunk_body(i, carry, _ch=ch, _buf=buf):
                ig = _ch * (CH // 256) + i
                a_splat = (_take(t01, zeros + (ig >> 4)) *
                           _take(t23, zeros + (ig & 15)))
                base = i * 256
                for j in range(16):
                    _buf[pl.ds(base + 16 * j, 16)] = a_splat * c_regs[j]
                return carry

            lax.fori_loop(0, CH // 256, chunk_body, 0)
            pending[p] = pltpu.async_copy(
                buf, o_hbm.at[b, pl.ds(ch * CH, CH)], sems[p])

    for p in range(2):
        if pending[p] is not None:
            pending[p].wait()


RB = 8


def _tc_body(x_ref, o_ref):
    x = x_ref[...]
    pio = lax.broadcasted_iota(jnp.int32, (S, 256), 1)
    sio = lax.broadcasted_iota(jnp.int32, (S, 256), 0)

    def fold(d0):
        acc = None
        for d in range(d0, d0 + 4):
            onehot = (((pio >> (2 * (d0 + 3 - d))) & 3) == sio)
            term = jnp.dot(x[:, :, d], onehot.astype(jnp.float32),
                           preferred_element_type=jnp.float32)
            acc = term if acc is None else acc * term
        return acc

    u = fold(0)
    v = fold(4)
    o_ref[...] = u[:, :, None] * v[:, None, :]


@jax.jit
def kernel(lmf, umf):
    lt = lmf.reshape(B, S * D)
    mesh = plsc.VectorSubcoreMesh(core_axis_name="c", subcore_axis_name="s")
    out_l = pl.kernel(
        _sc_body,
        out_type=jax.ShapeDtypeStruct((B, R), jnp.float32),
        mesh=mesh,
        scratch_types=[
            pltpu.VMEM((S * D,), jnp.float32),
            pltpu.VMEM((CH,), jnp.float32),
            pltpu.VMEM((CH,), jnp.float32),
            pltpu.SemaphoreType.DMA,
            pltpu.SemaphoreType.DMA,
        ],
    )(lt)
    out_u3 = pl.pallas_call(
        _tc_body,
        grid=(B // RB,),
        in_specs=[pl.BlockSpec((RB, S, D), lambda i: (i, 0, 0))],
        out_specs=pl.BlockSpec((RB, 256, 256), lambda i: (i, 0, 0)),
        out_shape=jax.ShapeDtypeStruct((B, 256, 256), jnp.float32),
    )(umf)
    return out_l, out_u3.reshape(B, R)


# speedup vs baseline: 1.6730x; 1.6730x over previous
import jax
import jax.numpy as jnp
from jax import lax
from jax.experimental import pallas as pl
from jax.experimental.pallas import tpu as pltpu
from jax.experimental.pallas import tpu_sc as plsc

B = 64
S = 4
D = 8
R = S ** D
NC = 2
NS = 16
NW = NC * NS
PER_W = 2 * B // NW

_DNUMS = lax.GatherDimensionNumbers(
    offset_dims=(), collapsed_slice_dims=(0,), start_index_map=(0,))


def _take(v, idx):
    return lax.gather(v, idx[:, None], _DNUMS, (1,),
                      mode=lax.GatherScatterMode.PROMISE_IN_BOUNDS)


CH = 8192
NCHUNK = R // CH


def _sc_body(l_hbm, out_l, in_v, buf0, buf1, sem0, sem1):
    cid = lax.axis_index("c")
    sid = lax.axis_index("s")
    wid = sid * NC + cid

    iota = lax.iota(jnp.int32, 16)
    hi = iota >> 2
    lo = iota & 3
    zeros = jnp.zeros((16,), jnp.int32)

    bufs = (buf0, buf1)
    sems = (sem0, sem1)
    pending = [None, None]
    gch = 0

    for x_hbm, o_hbm in ((l_hbm, out_l),):
      for k in range(B // NW):
        b = wid + NW * k

        pltpu.sync_copy(x_hbm.at[b], in_v)
        vlow = in_v[pl.ds(0, 16)]
        vhigh = in_v[pl.ds(16, 16)]

        def fac(sv, d):
            idx = ((sv & 1) << 3) + d
            return jnp.where(sv < 2, _take(vlow, idx), _take(vhigh, idx))

        t01 = fac(hi, 0) * fac(lo, 1)
        t23 = fac(hi, 2) * fac(lo, 3)
        t45 = fac(hi, 4) * fac(lo, 5)
        t67 = fac(hi, 6) * fac(lo, 7)

        c_regs = [_take(t45, jnp.full((16,), j, jnp.int32)) * t67
                  for j in range(16)]

        for ch in range(NCHUNK):
            p = gch & 1
            gch += 1
            buf = bufs[p]
            if pending[p] is not None:
                pending[p].wait()

            def chunk_body(i, carry, _ch=ch, _buf=buf):
                ig = _ch * (CH // 256) + i
                a_splat = (_take(t01, zeros + (ig >> 4)) *
                           _take(t23, zeros + (ig & 15)))
                base = i * 256
                for j in range(16):
                    _buf[pl.ds(base + 16 * j, 16)] = a_splat * c_regs[j]
                return carry

            lax.fori_loop(0, CH // 256, chunk_body, 0)
            pending[p] = pltpu.async_copy(
                buf, o_hbm.at[b, pl.ds(ch * CH, CH)], sems[p])

    for p in range(2):
        if pending[p] is not None:
            pending[p].wait()


RB = 8


def _tc_body(x_ref, o_ref):
    x = x_ref[...]
    pio = lax.broadcasted_iota(jnp.int32, (S, 256), 1)
    sio = lax.broadcasted_iota(jnp.int32, (S, 256), 0)

    def fold(d0):
        acc = None
        for d in range(d0, d0 + 4):
            onehot = (((pio >> (2 * (d0 + 3 - d))) & 3) == sio)
            term = jnp.dot(x[:, :, d], onehot.astype(jnp.float32),
                           preferred_element_type=jnp.float32,
                           precision=lax.Precision.HIGHEST)
            acc = term if acc is None else acc * term
        return acc

    u = fold(0)
    v = fold(4)
    for p in range(256):
        o_ref[:, pl.ds(256 * p, 256)] = u[:, p][:, None] * v


@jax.jit
def kernel(lmf, umf):
    lt = lmf.reshape(B, S * D)
    mesh = plsc.VectorSubcoreMesh(core_axis_name="c", subcore_axis_name="s")
    out_l = pl.kernel(
        _sc_body,
        out_type=jax.ShapeDtypeStruct((B, R), jnp.float32),
        mesh=mesh,
        scratch_types=[
            pltpu.VMEM((S * D,), jnp.float32),
            pltpu.VMEM((CH,), jnp.float32),
            pltpu.VMEM((CH,), jnp.float32),
            pltpu.SemaphoreType.DMA,
            pltpu.SemaphoreType.DMA,
        ],
    )(lt)
    out_u = pl.pallas_call(
        _tc_body,
        grid=(B // RB,),
        in_specs=[pl.BlockSpec((RB, S, D), lambda i: (i, 0, 0))],
        out_specs=pl.BlockSpec((RB, R), lambda i: (i, 0)),
        out_shape=jax.ShapeDtypeStruct((B, R), jnp.float32),
    )(umf)
    return out_l, out_u


# speedup vs baseline: 1.6825x; 1.0057x over previous
import jax
import jax.numpy as jnp
from jax import lax
from jax.experimental import pallas as pl
from jax.experimental.pallas import tpu as pltpu
from jax.experimental.pallas import tpu_sc as plsc

B = 64
S = 4
D = 8
R = S ** D
NC = 2
NS = 16
NW = NC * NS
PER_W = 2 * B // NW

_DNUMS = lax.GatherDimensionNumbers(
    offset_dims=(), collapsed_slice_dims=(0,), start_index_map=(0,))


def _take(v, idx):
    return lax.gather(v, idx[:, None], _DNUMS, (1,),
                      mode=lax.GatherScatterMode.PROMISE_IN_BOUNDS)


CH = 8192
NCHUNK = R // CH


def _sc_body(l_hbm, out_l, in_v, buf0, buf1, sem0, sem1):
    cid = lax.axis_index("c")
    sid = lax.axis_index("s")
    wid = sid * NC + cid

    iota = lax.iota(jnp.int32, 16)
    hi = iota >> 2
    lo = iota & 3
    zeros = jnp.zeros((16,), jnp.int32)

    bufs = (buf0, buf1)
    sems = (sem0, sem1)
    pending = [None, None]
    gch = 0

    for x_hbm, o_hbm in ((l_hbm, out_l),):
      for k in range(B // NW):
        b = wid + NW * k

        pltpu.sync_copy(x_hbm.at[b], in_v)
        vlow = in_v[pl.ds(0, 16)]
        vhigh = in_v[pl.ds(16, 16)]

        def fac(sv, d):
            idx = ((sv & 1) << 3) + d
            return jnp.where(sv < 2, _take(vlow, idx), _take(vhigh, idx))

        t01 = fac(hi, 0) * fac(lo, 1)
        t23 = fac(hi, 2) * fac(lo, 3)
        t45 = fac(hi, 4) * fac(lo, 5)
        t67 = fac(hi, 6) * fac(lo, 7)

        c_regs = [_take(t45, jnp.full((16,), j, jnp.int32)) * t67
                  for j in range(16)]

        for ch in range(NCHUNK):
            p = gch & 1
            gch += 1
            buf = bufs[p]
            if pending[p] is not None:
                pending[p].wait()

            def chunk_body(i, carry, _ch=ch, _buf=buf):
                ig = _ch * (CH // 256) + i
                a_splat = (_take(t01, zeros + (ig >> 4)) *
                           _take(t23, zeros + (ig & 15)))
                base = i * 256
                for j in range(16):
                    _buf[pl.ds(base + 16 * j, 16)] = a_splat * c_regs[j]
                return carry

            lax.fori_loop(0, CH // 256, chunk_body, 0)
            pending[p] = pltpu.async_copy(
                buf, o_hbm.at[b, pl.ds(ch * CH, CH)], sems[p])

    for p in range(2):
        if pending[p] is not None:
            pending[p].wait()


RB = 16


def _tc_body(x_ref, o_ref):
    x = x_ref[...]
    pio = lax.broadcasted_iota(jnp.int32, (S, 256), 1)
    sio = lax.broadcasted_iota(jnp.int32, (S, 256), 0)

    def fold(d0):
        acc = None
        for d in range(d0, d0 + 4):
            onehot = (((pio >> (2 * (d0 + 3 - d))) & 3) == sio)
            term = jnp.dot(x[:, :, d], onehot.astype(jnp.float32),
                           preferred_element_type=jnp.float32,
                           precision=lax.Precision.HIGHEST)
            acc = term if acc is None else acc * term
        return acc

    u = fold(0)
    v = fold(4)
    for p in range(256):
        o_ref[:, pl.ds(256 * p, 256)] = u[:, p][:, None] * v


@jax.jit
def kernel(lmf, umf):
    lt = lmf.reshape(B, S * D)
    mesh = plsc.VectorSubcoreMesh(core_axis_name="c", subcore_axis_name="s")
    out_l = pl.kernel(
        _sc_body,
        out_type=jax.ShapeDtypeStruct((B, R), jnp.float32),
        mesh=mesh,
        scratch_types=[
            pltpu.VMEM((S * D,), jnp.float32),
            pltpu.VMEM((CH,), jnp.float32),
            pltpu.VMEM((CH,), jnp.float32),
            pltpu.SemaphoreType.DMA,
            pltpu.SemaphoreType.DMA,
        ],
    )(lt)
    out_u = pl.pallas_call(
        _tc_body,
        grid=(B // RB,),
        in_specs=[pl.BlockSpec((RB, S, D), lambda i: (i, 0, 0))],
        out_specs=pl.BlockSpec((RB, R), lambda i: (i, 0)),
        out_shape=jax.ShapeDtypeStruct((B, R), jnp.float32),
    )(umf)
    return out_l, out_u
